# baseline (speedup 1.0000x reference)
import dataclasses
import functools

import jax
import jax.numpy as jnp
from jax import lax
from jax.experimental import pallas as pl
from jax.experimental.pallas import tpu as pltpu
from jax.experimental.pallas import tpu_sc as plsc

_NC = 2
_NS = 16
_NW = _NC * _NS
_D = 64
_L = 16
_CHUNK = 128
_TBL = 32768
_TSH = 14
_EPAD = 144
_RPAD = 80


def _tc_transpose_body(in_ref, out_ref):
    h = _TBL // 2
    a = in_ref[:, 0:h][...].T
    b = in_ref[:, h:_TBL][...].T
    out_ref[...] = jnp.concatenate([a, b], axis=1)


def _tc_transpose(ent_t):
    d, n = ent_t.shape
    n_blocks = (n + _TBL - 1) // _TBL
    return pl.pallas_call(
        _tc_transpose_body,
        grid=(n_blocks,),
        in_specs=[pl.BlockSpec((d, _TBL), lambda i: (0, i))],
        out_specs=pl.BlockSpec((_TBL // 2, 2 * d), lambda i: (i, 0)),
        out_shape=jax.ShapeDtypeStruct((n_blocks * (_TBL // 2), 2 * d),
                                       jnp.float32),
    )(ent_t)


def _sc_score(B):
    b_per_w = B // _NW
    n_chunks = b_per_w // _CHUNK
    n_groups = _CHUNK // _L
    mesh = plsc.VectorSubcoreMesh(core_axis_name="c", subcore_axis_name="s")

    cp = pltpu.CompilerParams(use_tc_tiling_on_sc=False)
    if "needs_layout_passes" in pltpu.CompilerParams.__dataclass_fields__:
        cp = dataclasses.replace(cp, needs_layout_passes=False)

    @functools.partial(
        pl.kernel,
        mesh=mesh,
        compiler_params=cp,
        out_type=jax.ShapeDtypeStruct((B,), jnp.float32),
        scratch_types=[
            pltpu.VMEM((b_per_w,), jnp.int32),
            pltpu.VMEM((b_per_w,), jnp.int32),
            pltpu.VMEM((b_per_w,), jnp.int32),
            pltpu.VMEM((b_per_w,), jnp.int32),
            pltpu.VMEM((b_per_w,), jnp.int32),
            pltpu.VMEM((_CHUNK, _D), jnp.float32),
            pltpu.VMEM((_CHUNK, _D), jnp.float32),
            pltpu.VMEM((_CHUNK, _D), jnp.float32),
            pltpu.VMEM((_CHUNK, _D), jnp.float32),
            pltpu.VMEM((1000, _D), jnp.float32),
            pltpu.VMEM((b_per_w,), jnp.float32),
            pltpu.SemaphoreType.DMA,
            pltpu.SemaphoreType.DMA,
            pltpu.SemaphoreType.DMA,
        ],
    )
    def score_kernel(ent_hbm, rel_hbm, hidx_hbm, tidx_hbm, ridx_hbm, out_hbm,
                     hi_v, ti_v, ri_v, hp_v, tp_v,
                     hrow0_v, trow0_v, hrow1_v, trow1_v, rtab_v,
                     out_v, sem0, sem1, semr):
        wid = lax.axis_index("s") * _NC + lax.axis_index("c")
        base = wid * b_per_w
        src = pl.ds(base, b_per_w)
        rel_cp = pltpu.async_copy(rel_hbm, rtab_v, semr)
        pltpu.sync_copy(hidx_hbm.at[src], hi_v)
        pltpu.sync_copy(tidx_hbm.at[src], ti_v)
        pltpu.sync_copy(ridx_hbm.at[src], ri_v)
        for s in range(b_per_w // _L):
            sl = pl.ds(s * _L, _L)
            hi = hi_v[sl]
            ti = ti_v[sl]
            hp_v[sl] = lax.bitwise_or(
                lax.bitwise_or(
                    lax.bitwise_and(hi, -_TBL),
                    lax.shift_left(lax.bitwise_and(hi, (1 << _TSH) - 1), 1)),
                lax.bitwise_and(lax.shift_right_logical(hi, _TSH), 1))
            tp_v[sl] = lax.bitwise_or(
                lax.bitwise_or(
                    lax.bitwise_and(ti, -_TBL),
                    lax.shift_left(lax.bitwise_and(ti, (1 << _TSH) - 1), 1)),
                lax.bitwise_and(lax.shift_right_logical(ti, _TSH), 1))

        iota = lax.iota(jnp.int32, _L)
        bufs = ((hrow0_v, trow0_v), (hrow1_v, trow1_v))

        def fire(c, buf, sem):
            csl = pl.ds(c * _CHUNK, _CHUNK)
            pltpu.async_copy(ent_hbm.at[hp_v.at[csl]], buf[0], sem)
            pltpu.async_copy(ent_hbm.at[tp_v.at[csl]], buf[1], sem)

        def compute(c, buf):
            hrow_v, trow_v = buf
            for g in range(n_groups):
                lanes = iota + g * _L
                rsl = pl.ds(c * _CHUNK + g * _L, _L)
                rrows = ri_v[rsl]
                acc = jnp.zeros((_L,), jnp.float32)
                for d in range(_D):
                    dv = jnp.full((_L,), d, jnp.int32)
                    hv = plsc.load_gather(hrow_v, [lanes, dv])
                    tv = plsc.load_gather(trow_v, [lanes, dv])
                    rv = plsc.load_gather(rtab_v, [rrows, dv])
                    acc = acc + jnp.abs(hv + rv - tv)
                out_v[rsl] = acc

        def drain(buf, sem):
            pltpu.make_async_copy(ent_hbm.at[pl.ds(0, _CHUNK)], buf[0], sem).wait()
            pltpu.make_async_copy(ent_hbm.at[pl.ds(0, _CHUNK)], buf[1], sem).wait()

        fire(0, bufs[0], sem0)
        fire(1, bufs[1], sem1)
        rel_cp.wait()

        @pl.loop(0, n_chunks, step=2)
        def _(c):
            drain(bufs[0], sem0)
            compute(c, bufs[0])

            @pl.when(c + 2 < n_chunks)
            def _():
                fire(c + 2, bufs[0], sem0)

            drain(bufs[1], sem1)
            compute(c + 1, bufs[1])

            @pl.when(c + 3 < n_chunks)
            def _():
                fire(c + 3, bufs[1], sem1)

        pltpu.sync_copy(out_v, out_hbm.at[pl.ds(base, b_per_w)])

    return score_kernel


def kernel(entity_emb, relation_emb, pos_h, pos_r, pos_t):
    B = pos_h.shape[0]
    ent_wide = _tc_transpose(entity_emb.T)
    ent_rows = ent_wide.reshape(ent_wide.shape[0] * 2, _D)
    return _sc_score(B)(ent_rows, relation_emb, pos_h, pos_t, pos_r)


# speedup vs baseline: 1.0042x; 1.0042x over previous
import dataclasses
import functools

import jax
import jax.numpy as jnp
from jax import lax
from jax.experimental import pallas as pl
from jax.experimental.pallas import tpu as pltpu
from jax.experimental.pallas import tpu_sc as plsc

_NC = 2
_NS = 16
_NW = _NC * _NS
_D = 64
_L = 16
_CHUNK = 128
_TBL = 32768
_TSH = 14
_EPAD = 144
_RPAD = 80


def _tc_transpose_body(in_ref, out_ref):
    h = _TBL // 2
    a = in_ref[:, 0:h][...].T
    b = in_ref[:, h:_TBL][...].T
    out_ref[...] = jnp.concatenate([a, b], axis=1)


def _tc_transpose(ent_t):
    d, n = ent_t.shape
    n_blocks = (n + _TBL - 1) // _TBL
    return pl.pallas_call(
        _tc_transpose_body,
        grid=(n_blocks,),
        in_specs=[pl.BlockSpec((d, _TBL), lambda i: (0, i))],
        out_specs=pl.BlockSpec((_TBL // 2, 2 * d), lambda i: (i, 0)),
        out_shape=jax.ShapeDtypeStruct((n_blocks * (_TBL // 2), 2 * d),
                                       jnp.float32),
    )(ent_t)


def _sc_score(B):
    b_per_w = B // _NW
    n_chunks = b_per_w // _CHUNK
    n_groups = _CHUNK // _L
    mesh = plsc.VectorSubcoreMesh(core_axis_name="c", subcore_axis_name="s")

    cp = pltpu.CompilerParams(use_tc_tiling_on_sc=False)
    if "needs_layout_passes" in pltpu.CompilerParams.__dataclass_fields__:
        cp = dataclasses.replace(cp, needs_layout_passes=False)

    @functools.partial(
        pl.kernel,
        mesh=mesh,
        compiler_params=cp,
        out_type=jax.ShapeDtypeStruct((B,), jnp.float32),
        scratch_types=[
            pltpu.VMEM((b_per_w,), jnp.int32),
            pltpu.VMEM((b_per_w,), jnp.int32),
            pltpu.VMEM((b_per_w,), jnp.int32),
            pltpu.VMEM((b_per_w,), jnp.int32),
            pltpu.VMEM((b_per_w,), jnp.int32),
            pltpu.VMEM((_CHUNK, _D), jnp.float32),
            pltpu.VMEM((_CHUNK, _D), jnp.float32),
            pltpu.VMEM((_CHUNK, _D), jnp.float32),
            pltpu.VMEM((_CHUNK, _D), jnp.float32),
            pltpu.VMEM((1000, _D), jnp.float32),
            pltpu.VMEM((b_per_w,), jnp.float32),
            pltpu.SemaphoreType.DMA,
            pltpu.SemaphoreType.DMA,
            pltpu.SemaphoreType.DMA,
        ],
    )
    def score_kernel(ent_hbm, rel_hbm, hidx_hbm, tidx_hbm, ridx_hbm, out_hbm,
                     hi_v, ti_v, ri_v, hp_v, tp_v,
                     hrow0_v, trow0_v, hrow1_v, trow1_v, rtab_v,
                     out_v, sem0, sem1, semr):
        wid = lax.axis_index("s") * _NC + lax.axis_index("c")
        base = wid * b_per_w
        src = pl.ds(base, b_per_w)
        rel_cp = pltpu.async_copy(rel_hbm, rtab_v, semr)
        pltpu.sync_copy(hidx_hbm.at[src], hi_v)
        pltpu.sync_copy(tidx_hbm.at[src], ti_v)
        pltpu.sync_copy(ridx_hbm.at[src], ri_v)
        for s in range(b_per_w // _L):
            sl = pl.ds(s * _L, _L)
            hi = hi_v[sl]
            ti = ti_v[sl]
            hp_v[sl] = lax.bitwise_or(
                lax.bitwise_or(
                    lax.bitwise_and(hi, -_TBL),
                    lax.shift_left(lax.bitwise_and(hi, (1 << _TSH) - 1), 1)),
                lax.bitwise_and(lax.shift_right_logical(hi, _TSH), 1))
            tp_v[sl] = lax.bitwise_or(
                lax.bitwise_or(
                    lax.bitwise_and(ti, -_TBL),
                    lax.shift_left(lax.bitwise_and(ti, (1 << _TSH) - 1), 1)),
                lax.bitwise_and(lax.shift_right_logical(ti, _TSH), 1))

        iota = lax.iota(jnp.int32, _L)
        bufs = ((hrow0_v, trow0_v), (hrow1_v, trow1_v))

        def fire(c, buf, sem):
            csl = pl.ds(c * _CHUNK, _CHUNK)
            pltpu.async_copy(ent_hbm.at[hp_v.at[csl]], buf[0], sem)
            pltpu.async_copy(ent_hbm.at[tp_v.at[csl]], buf[1], sem)

        def compute(c, buf):
            hrow_v, trow_v = buf

            @pl.loop(0, n_groups)
            def _(g):
                lanes = iota + g * _L
                rsl = pl.ds(c * _CHUNK + g * _L, _L)
                rrows = ri_v[rsl]
                out_v[rsl] = jnp.zeros((_L,), jnp.float32)

                @pl.loop(0, _D, step=8)
                def _(d0):
                    acc = jnp.zeros((_L,), jnp.float32)
                    for dd in range(8):
                        dv = jnp.zeros((_L,), jnp.int32) + (d0 + dd)
                        hv = plsc.load_gather(hrow_v, [lanes, dv])
                        tv = plsc.load_gather(trow_v, [lanes, dv])
                        rv = plsc.load_gather(rtab_v, [rrows, dv])
                        acc = acc + jnp.abs(hv + rv - tv)
                    plsc.addupdate(out_v.at[rsl], acc)

        def drain(buf, sem):
            pltpu.make_async_copy(ent_hbm.at[pl.ds(0, _CHUNK)], buf[0], sem).wait()
            pltpu.make_async_copy(ent_hbm.at[pl.ds(0, _CHUNK)], buf[1], sem).wait()

        fire(0, bufs[0], sem0)
        fire(1, bufs[1], sem1)
        rel_cp.wait()

        @pl.loop(0, n_chunks, step=2)
        def _(c):
            drain(bufs[0], sem0)
            compute(c, bufs[0])

            @pl.when(c + 2 < n_chunks)
            def _():
                fire(c + 2, bufs[0], sem0)

            drain(bufs[1], sem1)
            compute(c + 1, bufs[1])

            @pl.when(c + 3 < n_chunks)
            def _():
                fire(c + 3, bufs[1], sem1)

        pltpu.sync_copy(out_v, out_hbm.at[pl.ds(base, b_per_w)])

    return score_kernel


def kernel(entity_emb, relation_emb, pos_h, pos_r, pos_t):
    B = pos_h.shape[0]
    ent_wide = _tc_transpose(entity_emb.T)
    ent_rows = ent_wide.reshape(ent_wide.shape[0] * 2, _D)
    return _sc_score(B)(ent_rows, relation_emb, pos_h, pos_t, pos_r)


# speedup vs baseline: 1.0087x; 1.0045x over previous
import dataclasses
import functools

import jax
import jax.numpy as jnp
from jax import lax
from jax.experimental import pallas as pl
from jax.experimental.pallas import tpu as pltpu
from jax.experimental.pallas import tpu_sc as plsc

_NC = 2
_NS = 16
_NW = _NC * _NS
_D = 64
_L = 16
_CHUNK = 128
_TBL = 32768
_TSH = 14
_EPAD = 144
_RPAD = 80


def _tc_transpose_body(in_ref, out_ref):
    h = _TBL // 2
    a = in_ref[:, 0:h][...].T
    b = in_ref[:, h:_TBL][...].T
    out_ref[...] = jnp.concatenate([a, b], axis=1)


def _tc_transpose(ent_t):
    d, n = ent_t.shape
    n_blocks = (n + _TBL - 1) // _TBL
    return pl.pallas_call(
        _tc_transpose_body,
        grid=(n_blocks,),
        in_specs=[pl.BlockSpec((d, _TBL), lambda i: (0, i))],
        out_specs=pl.BlockSpec((_TBL // 2, 2 * d), lambda i: (i, 0)),
        out_shape=jax.ShapeDtypeStruct((n_blocks * (_TBL // 2), 2 * d),
                                       jnp.float32),
    )(ent_t)


def _sc_score(B):
    b_per_w = B // _NW
    n_chunks = b_per_w // _CHUNK
    n_groups = _CHUNK // _L
    mesh = plsc.VectorSubcoreMesh(core_axis_name="c", subcore_axis_name="s")

    cp = pltpu.CompilerParams(use_tc_tiling_on_sc=False)
    if "needs_layout_passes" in pltpu.CompilerParams.__dataclass_fields__:
        cp = dataclasses.replace(cp, needs_layout_passes=False)

    @functools.partial(
        pl.kernel,
        mesh=mesh,
        compiler_params=cp,
        out_type=jax.ShapeDtypeStruct((B,), jnp.float32),
        scratch_types=[
            pltpu.VMEM((b_per_w,), jnp.int32),
            pltpu.VMEM((b_per_w,), jnp.int32),
            pltpu.VMEM((b_per_w,), jnp.int32),
            pltpu.VMEM((b_per_w, _D), jnp.float32),
            pltpu.VMEM((b_per_w, _D), jnp.float32),
            pltpu.VMEM((b_per_w, _D), jnp.float32),
            pltpu.VMEM((b_per_w,), jnp.float32),
            pltpu.SemaphoreType.DMA,
        ],
    )
    def score_kernel(ent_hbm, rel_hbm, hidx_hbm, tidx_hbm, ridx_hbm, out_hbm,
                     hi_v, ti_v, ri_v,
                     hrow_v, trow_v, rrow_v, out_v, sem):
        wid = lax.axis_index("s") * _NC + lax.axis_index("c")
        base = wid * b_per_w
        src = pl.ds(base, b_per_w)
        pltpu.sync_copy(hidx_hbm.at[src], hi_v)
        pltpu.sync_copy(tidx_hbm.at[src], ti_v)
        pltpu.sync_copy(ridx_hbm.at[src], ri_v)
        for s in range(b_per_w // _L):
            sl = pl.ds(s * _L, _L)
            hi = hi_v[sl]
            ti = ti_v[sl]
            hi_v[sl] = lax.bitwise_or(
                lax.bitwise_or(
                    lax.bitwise_and(hi, -_TBL),
                    lax.shift_left(lax.bitwise_and(hi, (1 << _TSH) - 1), 1)),
                lax.bitwise_and(lax.shift_right_logical(hi, _TSH), 1))
            ti_v[sl] = lax.bitwise_or(
                lax.bitwise_or(
                    lax.bitwise_and(ti, -_TBL),
                    lax.shift_left(lax.bitwise_and(ti, (1 << _TSH) - 1), 1)),
                lax.bitwise_and(lax.shift_right_logical(ti, _TSH), 1))

        iota = lax.iota(jnp.int32, _L)

        copies = []
        for c in range(n_chunks):
            csl = pl.ds(c * _CHUNK, _CHUNK)
            dst = pl.ds(c * _CHUNK, _CHUNK)
            copies.append(pltpu.async_copy(
                ent_hbm.at[hi_v.at[csl]], hrow_v.at[dst], sem))
            copies.append(pltpu.async_copy(
                ent_hbm.at[ti_v.at[csl]], trow_v.at[dst], sem))
            copies.append(pltpu.async_copy(
                rel_hbm.at[ri_v.at[csl]], rrow_v.at[dst], sem))
        for cpy in copies:
            cpy.wait()

        @pl.loop(0, b_per_w // _L)
        def _(g):
            lanes = iota + g * _L
            rsl = pl.ds(g * _L, _L)
            out_v[rsl] = jnp.zeros((_L,), jnp.float32)

            @pl.loop(0, _D, step=8)
            def _(d0):
                acc = jnp.zeros((_L,), jnp.float32)
                for dd in range(8):
                    dv = jnp.zeros((_L,), jnp.int32) + (d0 + dd)
                    hv = plsc.load_gather(hrow_v, [lanes, dv])
                    tv = plsc.load_gather(trow_v, [lanes, dv])
                    rv = plsc.load_gather(rrow_v, [lanes, dv])
                    acc = acc + jnp.abs(hv + rv - tv)
                plsc.addupdate(out_v.at[rsl], acc)

        pltpu.sync_copy(out_v, out_hbm.at[pl.ds(base, b_per_w)])

    return score_kernel


def kernel(entity_emb, relation_emb, pos_h, pos_r, pos_t):
    B = pos_h.shape[0]
    ent_wide = _tc_transpose(entity_emb.T)
    ent_rows = ent_wide.reshape(ent_wide.shape[0] * 2, _D)
    return _sc_score(B)(ent_rows, relation_emb, pos_h, pos_t, pos_r)
